# baseline (device time: 207328 ns/iter reference)
import jax
import jax.numpy as jnp
from jax import lax
from jax.experimental import pallas as pl
from jax.experimental.pallas import tpu as pltpu

N_DEV = 16
M_BLK = 128
N_COL = 2048


def kernel(x, w_mat):
    m_blk, n_col = M_BLK, N_COL

    def body(x_ref, w_ref, out_ref, send_buf, recv_buf, send_sems, recv_sems,
             credit_sem):
        me = lax.axis_index("i")
        left = (me - 1) % N_DEV
        right = (me + 1) % N_DEV

        barrier_sem = pltpu.get_barrier_semaphore()
        for nbr in [left, right]:
            pl.semaphore_signal(
                barrier_sem, inc=1,
                device_id=(nbr,), device_id_type=pl.DeviceIdType.MESH,
            )
        pl.semaphore_wait(barrier_sem, 2)

        def partial(c):
            xc = x_ref[pl.ds(c * m_blk, m_blk), :]
            return jnp.dot(xc, w_ref[:, :], preferred_element_type=jnp.float32)

        send_buf[0, :, :] = partial((me - 1) % N_DEV)

        for t in range(N_DEV - 1):
            slot = t % 2
            if t >= 2:
                pl.semaphore_wait(credit_sem, 1)
            rdma = pltpu.make_async_remote_copy(
                src_ref=send_buf.at[slot],
                dst_ref=recv_buf.at[slot],
                send_sem=send_sems.at[slot],
                recv_sem=recv_sems.at[slot],
                device_id=(right,),
                device_id_type=pl.DeviceIdType.MESH,
            )
            rdma.start()
            rdma.wait()

            c_recv = (me - 2 - t) % N_DEV
            acc = recv_buf[slot, :, :] + partial(c_recv)
            if t < N_DEV - 2:
                send_buf[(t + 1) % 2, :, :] = acc
                if t <= N_DEV - 4:
                    pl.semaphore_signal(
                        credit_sem, inc=1,
                        device_id=(left,), device_id_type=pl.DeviceIdType.MESH,
                    )
            else:
                out_ref[:, :] = acc * jax.nn.sigmoid(acc)

    return pl.pallas_call(
        body,
        out_shape=jax.ShapeDtypeStruct((m_blk, n_col), jnp.float32),
        in_specs=[
            pl.BlockSpec(memory_space=pltpu.VMEM),
            pl.BlockSpec(memory_space=pltpu.VMEM),
        ],
        out_specs=pl.BlockSpec(memory_space=pltpu.VMEM),
        scratch_shapes=[
            pltpu.VMEM((2, m_blk, n_col), jnp.float32),
            pltpu.VMEM((2, m_blk, n_col), jnp.float32),
            pltpu.SemaphoreType.DMA((2,)),
            pltpu.SemaphoreType.DMA((2,)),
            pltpu.SemaphoreType.REGULAR,
        ],
        compiler_params=pltpu.CompilerParams(collective_id=0),
    )(x, w_mat)


# device time: 140575 ns/iter; 1.4749x vs baseline; 1.4749x over previous
import jax
import jax.numpy as jnp
from jax import lax
from jax.experimental import pallas as pl
from jax.experimental.pallas import tpu as pltpu

N_DEV = 16
M_BLK = 128
N_COL = 2048
N_HALF = N_COL // 2


def kernel(x, w_mat):
    def body(x_ref, w_ref, out_ref,
             send_a, recv_a, send_b, recv_b,
             send_sems_a, recv_sems_a, send_sems_b, recv_sems_b,
             credit_a, credit_b):
        me = lax.axis_index("i")
        left = (me - 1) % N_DEV
        right = (me + 1) % N_DEV

        barrier_sem = pltpu.get_barrier_semaphore()
        for nbr in [left, right]:
            pl.semaphore_signal(
                barrier_sem, inc=1,
                device_id=(nbr,), device_id_type=pl.DeviceIdType.MESH,
            )
        pl.semaphore_wait(barrier_sem, 2)

        def partial_a(c):
            xc = x_ref[pl.ds(c * M_BLK, M_BLK), :]
            return jnp.dot(xc, w_ref[:, :N_HALF],
                           preferred_element_type=jnp.float32)

        def partial_b(c):
            xc = x_ref[pl.ds(c * M_BLK, M_BLK), :]
            return jnp.dot(xc, w_ref[:, N_HALF:],
                           preferred_element_type=jnp.float32)

        def silu(y):
            return y * jax.nn.sigmoid(y)

        send_a[0, :, :] = partial_a((me - 1) % N_DEV)
        send_b[0, :, :] = partial_b((me + 1) % N_DEV)

        for t in range(N_DEV - 1):
            slot = t % 2
            if t >= 2:
                pl.semaphore_wait(credit_a, 1)
                pl.semaphore_wait(credit_b, 1)
            rdma_a = pltpu.make_async_remote_copy(
                src_ref=send_a.at[slot],
                dst_ref=recv_a.at[slot],
                send_sem=send_sems_a.at[slot],
                recv_sem=recv_sems_a.at[slot],
                device_id=(right,),
                device_id_type=pl.DeviceIdType.MESH,
            )
            rdma_b = pltpu.make_async_remote_copy(
                src_ref=send_b.at[slot],
                dst_ref=recv_b.at[slot],
                send_sem=send_sems_b.at[slot],
                recv_sem=recv_sems_b.at[slot],
                device_id=(left,),
                device_id_type=pl.DeviceIdType.MESH,
            )
            rdma_a.start()
            rdma_b.start()

            c_recv_a = (me - 2 - t) % N_DEV
            c_recv_b = (me + 2 + t) % N_DEV
            p_a = partial_a(c_recv_a)
            p_b = partial_b(c_recv_b)

            rdma_a.wait()
            rdma_b.wait()

            acc_a = recv_a[slot, :, :] + p_a
            acc_b = recv_b[slot, :, :] + p_b
            if t < N_DEV - 2:
                send_a[(t + 1) % 2, :, :] = acc_a
                send_b[(t + 1) % 2, :, :] = acc_b
                if t <= N_DEV - 4:
                    pl.semaphore_signal(
                        credit_a, inc=1,
                        device_id=(left,), device_id_type=pl.DeviceIdType.MESH,
                    )
                    pl.semaphore_signal(
                        credit_b, inc=1,
                        device_id=(right,), device_id_type=pl.DeviceIdType.MESH,
                    )
            else:
                out_ref[:, :N_HALF] = silu(acc_a)
                out_ref[:, N_HALF:] = silu(acc_b)

    return pl.pallas_call(
        body,
        out_shape=jax.ShapeDtypeStruct((M_BLK, N_COL), jnp.float32),
        in_specs=[
            pl.BlockSpec(memory_space=pltpu.VMEM),
            pl.BlockSpec(memory_space=pltpu.VMEM),
        ],
        out_specs=pl.BlockSpec(memory_space=pltpu.VMEM),
        scratch_shapes=[
            pltpu.VMEM((2, M_BLK, N_HALF), jnp.float32),
            pltpu.VMEM((2, M_BLK, N_HALF), jnp.float32),
            pltpu.VMEM((2, M_BLK, N_HALF), jnp.float32),
            pltpu.VMEM((2, M_BLK, N_HALF), jnp.float32),
            pltpu.SemaphoreType.DMA((2,)),
            pltpu.SemaphoreType.DMA((2,)),
            pltpu.SemaphoreType.DMA((2,)),
            pltpu.SemaphoreType.DMA((2,)),
            pltpu.SemaphoreType.REGULAR,
            pltpu.SemaphoreType.REGULAR,
        ],
        compiler_params=pltpu.CompilerParams(collective_id=0),
    )(x, w_mat)


# device time: 96540 ns/iter; 2.1476x vs baseline; 1.4561x over previous
import jax
import jax.numpy as jnp
from jax import lax
from jax.experimental import pallas as pl
from jax.experimental.pallas import tpu as pltpu

N_DEV = 16
N_HOP = N_DEV - 1
M_BLK = 128
N_COL = 2048
N_HALF = N_COL // 2
SUB = 2
SLOTS = 2
N_SUB = N_HALF // SUB


def kernel(x, w_mat):
    def body(x_ref, w_ref, out_ref,
             send_a, recv_a, send_b, recv_b,
             send_sems_a, recv_sems_a, send_sems_b, recv_sems_b,
             credit_a, credit_b):
        me = lax.axis_index("i")
        left = (me - 1) % N_DEV
        right = (me + 1) % N_DEV

        barrier_sem = pltpu.get_barrier_semaphore()
        for nbr in [left, right]:
            pl.semaphore_signal(
                barrier_sem, inc=1,
                device_id=(nbr,), device_id_type=pl.DeviceIdType.MESH,
            )
        pl.semaphore_wait(barrier_sem, 2)

        rings = {
            "a": dict(send=send_a, recv=recv_a, ssem=send_sems_a,
                      rsem=recv_sems_a, credit=credit_a, to=right,
                      frm=left, col0=0),
            "b": dict(send=send_b, recv=recv_b, ssem=send_sems_b,
                      rsem=recv_sems_b, credit=credit_b, to=left,
                      frm=right, col0=N_HALF),
        }

        def partial(c, ring, s):
            xc = x_ref[pl.ds(c * M_BLK, M_BLK), :]
            lo = rings[ring]["col0"] + s * N_SUB
            return jnp.dot(xc, w_ref[:, lo:lo + N_SUB],
                           preferred_element_type=jnp.float32)

        def c_send(t, ring):
            return (me - 1 - t) % N_DEV if ring == "a" else (me + 1 + t) % N_DEV

        def silu(y):
            return y * jax.nn.sigmoid(y)

        rdmas = {}

        def start_send(t, ring, s):
            r = rings[ring]
            if t >= SLOTS:
                pl.semaphore_wait(r["credit"].at[s], 1)
            d = pltpu.make_async_remote_copy(
                src_ref=r["send"].at[t % SLOTS, s],
                dst_ref=r["recv"].at[t % SLOTS, s],
                send_sem=r["ssem"].at[t % SLOTS, s],
                recv_sem=r["rsem"].at[t % SLOTS, s],
                device_id=(r["to"],),
                device_id_type=pl.DeviceIdType.MESH,
            )
            rdmas[(t, ring, s)] = d
            d.start()

        for ring in ("a", "b"):
            for s in range(SUB):
                rings[ring]["send"][0, s, :, :] = partial(c_send(0, ring), ring, s)
        for s in range(SUB):
            for ring in ("a", "b"):
                start_send(0, ring, s)

        unwaited_sends = set(rdmas.keys())

        for t in range(N_HOP):
            for s in range(SUB):
                for ring in ("a", "b"):
                    r = rings[ring]
                    c_in = c_send(t + 1, ring)
                    p = partial(c_in, ring, s)
                    rdmas[(t, ring, s)].wait_recv()
                    acc = r["recv"][t % SLOTS, s, :, :] + p
                    if t < N_HOP - 1:
                        if t + 1 - SLOTS >= 0:
                            key = (t + 1 - SLOTS, ring, s)
                            rdmas[key].wait_send()
                            unwaited_sends.discard(key)
                        r["send"][(t + 1) % SLOTS, s, :, :] = acc
                        if t <= N_HOP - 1 - SLOTS:
                            pl.semaphore_signal(
                                r["credit"].at[s], inc=1,
                                device_id=(r["frm"],),
                                device_id_type=pl.DeviceIdType.MESH,
                            )
                        start_send(t + 1, ring, s)
                        unwaited_sends.add((t + 1, ring, s))
                    else:
                        lo = r["col0"] + s * N_SUB
                        out_ref[:, lo:lo + N_SUB] = silu(acc)

        for key in sorted(unwaited_sends):
            rdmas[key].wait_send()

    return pl.pallas_call(
        body,
        out_shape=jax.ShapeDtypeStruct((M_BLK, N_COL), jnp.float32),
        in_specs=[
            pl.BlockSpec(memory_space=pltpu.VMEM),
            pl.BlockSpec(memory_space=pltpu.VMEM),
        ],
        out_specs=pl.BlockSpec(memory_space=pltpu.VMEM),
        scratch_shapes=[
            pltpu.VMEM((SLOTS, SUB, M_BLK, N_SUB), jnp.float32),
            pltpu.VMEM((SLOTS, SUB, M_BLK, N_SUB), jnp.float32),
            pltpu.VMEM((SLOTS, SUB, M_BLK, N_SUB), jnp.float32),
            pltpu.VMEM((SLOTS, SUB, M_BLK, N_SUB), jnp.float32),
            pltpu.SemaphoreType.DMA((SLOTS, SUB)),
            pltpu.SemaphoreType.DMA((SLOTS, SUB)),
            pltpu.SemaphoreType.DMA((SLOTS, SUB)),
            pltpu.SemaphoreType.DMA((SLOTS, SUB)),
            pltpu.SemaphoreType.REGULAR((SUB,)),
            pltpu.SemaphoreType.REGULAR((SUB,)),
        ],
        compiler_params=pltpu.CompilerParams(collective_id=0),
    )(x, w_mat)
